# Initial kernel scaffold; baseline (speedup 1.0000x reference)
import jax
import jax.numpy as jnp
from jax.experimental import pallas as pl


def kernel(X, emb):
    raise NotImplementedError("write your pallas kernel here")


# speedup vs baseline: 1.2878x; 1.2878x over previous
import jax
import jax.numpy as jnp
from jax.experimental import pallas as pl

D_MODEL_ = 1024
WINDOW_ = 4096
BATCH_ = 4
BW = 256


def _add_kernel(x_ref, emb_ref, o_ref):
    o_ref[...] = x_ref[...] + emb_ref[...]


def kernel(X, emb):
    grid = (WINDOW_ // BW, BATCH_)
    return pl.pallas_call(
        _add_kernel,
        grid=grid,
        in_specs=[
            pl.BlockSpec((1, BW, D_MODEL_), lambda w, b: (b, w, 0)),
            pl.BlockSpec((BW, D_MODEL_), lambda w, b: (w, 0)),
        ],
        out_specs=pl.BlockSpec((1, BW, D_MODEL_), lambda w, b: (b, w, 0)),
        out_shape=jax.ShapeDtypeStruct(X.shape, X.dtype),
    )(X, emb)
